# baseline (speedup 1.0000x reference)
import functools

import jax
import jax.numpy as jnp
from jax import lax
from jax.experimental import pallas as pl
from jax.experimental.pallas import tpu as pltpu
from jax.experimental.pallas import tpu_sc as plsc

NC = 2
NS = 16
L = 16
NW = NC * NS
C = 128


def _pad_rows(n):
    return -(-n // (NS * 128)) * NS * 128


@functools.lru_cache(maxsize=None)
def _make_deg_kernel(EP, N):
    EPT = EP // NW
    CH = EPT // C
    NP = _pad_rows(N)
    RT = NP // NS
    ZR = 128
    W = 128
    assert EPT % C == 0 and RT % ZR == 0

    mesh = plsc.VectorSubcoreMesh(core_axis_name="c", subcore_axis_name="s")

    @functools.partial(
        pl.kernel,
        out_type=jax.ShapeDtypeStruct((NC * NP, W), jnp.float32),
        mesh=mesh,
        scratch_types=[
            pltpu.VMEM((1, C), jnp.int32),
            pltpu.VMEM((C, W), jnp.float32),
            pltpu.VMEM_SHARED((NP, W), jnp.float32),
        ],
    )
    def deg_kernel(dst_hbm, degp_hbm, didx, ones_v, deg_sh):
        c = lax.axis_index("c")
        s = lax.axis_index("s")
        zero16 = jnp.zeros((L,), jnp.float32)
        one16 = jnp.ones((L,), jnp.float32)
        G = W // L

        def fz(i, _):
            ones_v[i // G, pl.ds((i % G) * L, L)] = zero16
            return 0

        lax.fori_loop(0, ZR * G, fz, 0)

        def zc(k, _):
            pltpu.sync_copy(ones_v.at[pl.ds(0, ZR)],
                            deg_sh.at[pl.ds(s * RT + k * ZR, ZR)])
            return 0

        lax.fori_loop(0, RT // ZR, zc, 0)

        def fo(i, _):
            ones_v[i // G, pl.ds((i % G) * L, L)] = one16
            return 0

        lax.fori_loop(0, C * G, fo, 0)
        plsc.subcore_barrier()

        base = (c * NS + s) * EPT

        def chunk(i, _):
            pltpu.sync_copy(dst_hbm.at[pl.ds(base + i * C, C)], didx.at[0])
            pltpu.sync_copy(ones_v, deg_sh.at[didx.at[0]], add=True)
            return 0

        lax.fori_loop(0, CH, chunk, 0)
        plsc.subcore_barrier()

        def wb(k, _):
            pltpu.sync_copy(
                deg_sh.at[pl.ds(s * RT + k * ZR, ZR)],
                degp_hbm.at[pl.ds(c * NP + s * RT + k * ZR, ZR)],
            )
            return 0

        lax.fori_loop(0, RT // ZR, wb, 0)

    return deg_kernel


@functools.lru_cache(maxsize=None)
def _make_agg_kernel(EP, N, D):
    EPT = EP // NW
    CH = EPT // C
    NP = _pad_rows(N)
    RT = NP // NS
    ZR = 128
    G = D // L
    assert EPT % C == 0 and RT % ZR == 0 and D % L == 0

    mesh = plsc.VectorSubcoreMesh(core_axis_name="c", subcore_axis_name="s")

    @functools.partial(
        pl.kernel,
        out_type=jax.ShapeDtypeStruct((NC * NP, D), jnp.float32),
        mesh=mesh,
        scratch_types=[
            pltpu.VMEM((1, C), jnp.int32),
            pltpu.VMEM((1, C), jnp.int32),
            pltpu.VMEM((C, D), jnp.float32),
            pltpu.VMEM((ZR, D), jnp.float32),
            pltpu.VMEM_SHARED((NP, D), jnp.float32),
            pltpu.SemaphoreType.DMA,
        ],
    )
    def agg_kernel(y_hbm, src_hbm, dst_hbm, aggp_hbm,
                   sidx, didx, rows_v, zbuf_v, agg_sh, sem):
        c = lax.axis_index("c")
        s = lax.axis_index("s")
        zero16 = jnp.zeros((L,), jnp.float32)

        def fz(i, _):
            zbuf_v[i // G, pl.ds((i % G) * L, L)] = zero16
            return 0

        lax.fori_loop(0, ZR * G, fz, 0)

        def zc(k, _):
            pltpu.sync_copy(zbuf_v, agg_sh.at[pl.ds(s * RT + k * ZR, ZR)])
            return 0

        lax.fori_loop(0, RT // ZR, zc, 0)
        plsc.subcore_barrier()

        base = (c * NS + s) * EPT

        def chunk(i, _):
            pltpu.sync_copy(src_hbm.at[pl.ds(base + i * C, C)], sidx.at[0])
            pltpu.sync_copy(dst_hbm.at[pl.ds(base + i * C, C)], didx.at[0])
            pltpu.async_copy(y_hbm.at[sidx.at[0]], rows_v, sem).wait()
            pltpu.sync_copy(rows_v, agg_sh.at[didx.at[0]], add=True)
            return 0

        lax.fori_loop(0, CH, chunk, 0)
        plsc.subcore_barrier()

        def wb(k, _):
            pltpu.sync_copy(
                agg_sh.at[pl.ds(s * RT + k * ZR, ZR)],
                aggp_hbm.at[pl.ds(c * NP + s * RT + k * ZR, ZR)],
            )
            return 0

        lax.fori_loop(0, RT // ZR, wb, 0)

    return agg_kernel


_BR = 1000


def _mm_scale(x, W, deg2):
    N, Din = x.shape
    Dout = W.shape[1]

    def body(x_ref, w_ref, d_ref, o_ref):
        dis = lax.rsqrt(d_ref[...])
        o_ref[...] = dis * jnp.dot(
            x_ref[...], w_ref[...], preferred_element_type=jnp.float32
        )

    return pl.pallas_call(
        body,
        grid=(N // _BR,),
        in_specs=[
            pl.BlockSpec((_BR, Din), lambda i: (i, 0)),
            pl.BlockSpec((Din, Dout), lambda i: (0, 0)),
            pl.BlockSpec((_BR, 1), lambda i: (i, 0)),
        ],
        out_specs=pl.BlockSpec((_BR, Dout), lambda i: (i, 0)),
        out_shape=jax.ShapeDtypeStruct((N, Dout), jnp.float32),
    )(x, W, deg2)


def _mid(aggp, y1, deg2, b1, W2):
    N, D = y1.shape
    Dout = W2.shape[1]

    def body(a_ref, y_ref, d_ref, b_ref, w_ref, o_ref):
        dis = lax.rsqrt(d_ref[...])
        h = jnp.maximum(
            dis * (a_ref[0] + a_ref[1] + y_ref[...]) + b_ref[...], 0.0
        )
        o_ref[...] = dis * jnp.dot(
            h, w_ref[...], preferred_element_type=jnp.float32
        )

    return pl.pallas_call(
        body,
        grid=(N // _BR,),
        in_specs=[
            pl.BlockSpec((NC, _BR, D), lambda i: (0, i, 0)),
            pl.BlockSpec((_BR, D), lambda i: (i, 0)),
            pl.BlockSpec((_BR, 1), lambda i: (i, 0)),
            pl.BlockSpec((1, D), lambda i: (0, 0)),
            pl.BlockSpec((D, Dout), lambda i: (0, 0)),
        ],
        out_specs=pl.BlockSpec((_BR, Dout), lambda i: (i, 0)),
        out_shape=jax.ShapeDtypeStruct((N, Dout), jnp.float32),
    )(aggp, y1, deg2, b1, W2)


def _fin(aggp, y2, deg2, b2):
    N, D = y2.shape

    def body(a_ref, y_ref, d_ref, b_ref, o_ref):
        dis = lax.rsqrt(d_ref[...])
        o_ref[...] = dis * (a_ref[0] + a_ref[1] + y_ref[...]) + b_ref[...]

    return pl.pallas_call(
        body,
        grid=(N // _BR,),
        in_specs=[
            pl.BlockSpec((NC, _BR, D), lambda i: (0, i, 0)),
            pl.BlockSpec((_BR, D), lambda i: (i, 0)),
            pl.BlockSpec((_BR, 1), lambda i: (i, 0)),
            pl.BlockSpec((1, D), lambda i: (0, 0)),
        ],
        out_specs=pl.BlockSpec((_BR, D), lambda i: (i, 0)),
        out_shape=jax.ShapeDtypeStruct((N, D), jnp.float32),
    )(aggp, y2, deg2, b2)


def kernel(x, edge_index, W1, b1, W2, b2):
    N, _ = x.shape
    E = edge_index.shape[1]
    D = W1.shape[1]
    NP = _pad_rows(N)

    src = edge_index[0].astype(jnp.int32)
    dst = edge_index[1].astype(jnp.int32)

    EP = 2 * NW * C * (-(-E // (2 * NW * C)))
    if EP != E:
        pad = EP - E
        src = jnp.concatenate([src, jnp.zeros((pad,), jnp.int32)])
        dst = jnp.concatenate([dst, jnp.full((pad,), NP - 1, jnp.int32)])
    degp = _make_deg_kernel(EP, N)(dst)
    deg = degp.reshape(NC, NP, 128)[:, :N, 0].sum(0) + 1.0
    deg2 = deg.reshape(N, 1)

    y1 = _mm_scale(x, W1, deg2)
    agg1 = _make_agg_kernel(EP, N, D)(y1, src, dst).reshape(NC, NP, D)
    y2 = _mid(agg1, y1, deg2, b1.reshape(1, D), W2)
    agg2 = _make_agg_kernel(EP, N, D)(y2, src, dst).reshape(NC, NP, D)
    return _fin(agg2, y2, deg2, b2.reshape(1, D))


# speedup vs baseline: 1.2896x; 1.2896x over previous
import functools

import jax
import jax.numpy as jnp
from jax import lax
from jax.experimental import pallas as pl
from jax.experimental.pallas import tpu as pltpu
from jax.experimental.pallas import tpu_sc as plsc

NC = 2
NS = 16
L = 16
NW = NC * NS
C = 128


def _pad_rows(n):
    return -(-n // (NS * 128)) * NS * 128


@functools.lru_cache(maxsize=None)
def _make_deg_kernel(EP, N):
    EPT = EP // NW
    CH = EPT // C
    NP = _pad_rows(N)
    RT = NP // NS
    ZR = 128
    W = 128
    assert EPT % C == 0 and RT % ZR == 0

    mesh = plsc.VectorSubcoreMesh(core_axis_name="c", subcore_axis_name="s")

    @functools.partial(
        pl.kernel,
        out_type=jax.ShapeDtypeStruct((NC * NP, W), jnp.float32),
        mesh=mesh,
        scratch_types=[
            pltpu.VMEM((1, C), jnp.int32),
            pltpu.VMEM((C, W), jnp.float32),
            pltpu.VMEM_SHARED((NP, W), jnp.float32),
        ],
    )
    def deg_kernel(dst_hbm, degp_hbm, didx, ones_v, deg_sh):
        c = lax.axis_index("c")
        s = lax.axis_index("s")
        zero16 = jnp.zeros((L,), jnp.float32)
        one16 = jnp.ones((L,), jnp.float32)
        G = W // L

        def fz(i, _):
            ones_v[i // G, pl.ds((i % G) * L, L)] = zero16
            return 0

        lax.fori_loop(0, ZR * G, fz, 0)

        def zc(k, _):
            pltpu.sync_copy(ones_v.at[pl.ds(0, ZR)],
                            deg_sh.at[pl.ds(s * RT + k * ZR, ZR)])
            return 0

        lax.fori_loop(0, RT // ZR, zc, 0)

        def fo(i, _):
            ones_v[i // G, pl.ds((i % G) * L, L)] = one16
            return 0

        lax.fori_loop(0, C * G, fo, 0)
        plsc.subcore_barrier()

        base = (c * NS + s) * EPT

        def chunk(i, _):
            pltpu.sync_copy(dst_hbm.at[pl.ds(base + i * C, C)], didx.at[0])
            pltpu.sync_copy(ones_v, deg_sh.at[didx.at[0]], add=True)
            return 0

        lax.fori_loop(0, CH, chunk, 0)
        plsc.subcore_barrier()

        def wb(k, _):
            pltpu.sync_copy(
                deg_sh.at[pl.ds(s * RT + k * ZR, ZR)],
                degp_hbm.at[pl.ds(c * NP + s * RT + k * ZR, ZR)],
            )
            return 0

        lax.fori_loop(0, RT // ZR, wb, 0)

    return deg_kernel


@functools.lru_cache(maxsize=None)
def _make_agg_kernel(EP, N, D):
    EPT = EP // NW
    CH = EPT // C
    NP = _pad_rows(N)
    RT = NP // NS
    ZR = 128
    G = D // L
    assert EPT % C == 0 and RT % ZR == 0 and D % L == 0

    mesh = plsc.VectorSubcoreMesh(core_axis_name="c", subcore_axis_name="s")

    @functools.partial(
        pl.kernel,
        out_type=jax.ShapeDtypeStruct((NC * NP, D), jnp.float32),
        mesh=mesh,
        scratch_types=[
            pltpu.VMEM((1, C), jnp.int32),
            pltpu.VMEM((1, C), jnp.int32),
            pltpu.VMEM((C, D), jnp.float32),
            pltpu.VMEM((ZR, D), jnp.float32),
            pltpu.VMEM_SHARED((NP, D), jnp.float32),
            pltpu.SemaphoreType.DMA,
        ],
    )
    def agg_kernel(y_hbm, src_hbm, dst_hbm, aggp_hbm,
                   sidx, didx, rows_v, zbuf_v, agg_sh, sem):
        c = lax.axis_index("c")
        s = lax.axis_index("s")
        zero16 = jnp.zeros((L,), jnp.float32)

        def fz(i, _):
            zbuf_v[i // G, pl.ds((i % G) * L, L)] = zero16
            return 0

        lax.fori_loop(0, ZR * G, fz, 0)

        def zc(k, _):
            pltpu.sync_copy(zbuf_v, agg_sh.at[pl.ds(s * RT + k * ZR, ZR)])
            return 0

        lax.fori_loop(0, RT // ZR, zc, 0)
        plsc.subcore_barrier()

        base = (c * NS + s) * EPT

        def chunk(i, _):
            pltpu.sync_copy(src_hbm.at[pl.ds(base + i * C, C)], sidx.at[0])
            pltpu.sync_copy(dst_hbm.at[pl.ds(base + i * C, C)], didx.at[0])
            pltpu.async_copy(y_hbm.at[sidx.at[0]], rows_v, sem).wait()
            pltpu.sync_copy(rows_v, agg_sh.at[didx.at[0]], add=True)
            return 0

        lax.fori_loop(0, CH, chunk, 0)
        plsc.subcore_barrier()

        def wb(k, _):
            pltpu.sync_copy(
                agg_sh.at[pl.ds(s * RT + k * ZR, ZR)],
                aggp_hbm.at[pl.ds(c * NP + s * RT + k * ZR, ZR)],
            )
            return 0

        lax.fori_loop(0, RT // ZR, wb, 0)

    return agg_kernel


_BR = 1000


def _mm_scale(x, W, deg2):
    N, Din = x.shape
    Dout = W.shape[1]

    def body(x_ref, w_ref, d_ref, o_ref):
        dis = lax.rsqrt(d_ref[...])
        o_ref[...] = dis * jnp.dot(
            x_ref[...], w_ref[...], preferred_element_type=jnp.float32
        )

    return pl.pallas_call(
        body,
        grid=(N // _BR,),
        in_specs=[
            pl.BlockSpec((_BR, Din), lambda i: (i, 0)),
            pl.BlockSpec((Din, Dout), lambda i: (0, 0)),
            pl.BlockSpec((_BR, 1), lambda i: (i, 0)),
        ],
        out_specs=pl.BlockSpec((_BR, Dout), lambda i: (i, 0)),
        out_shape=jax.ShapeDtypeStruct((N, Dout), jnp.float32),
    )(x, W, deg2)


def _mid(aggp, y1, deg2, b1, W2):
    N, D = y1.shape
    Dout = W2.shape[1]

    def body(a_ref, y_ref, d_ref, b_ref, w_ref, o_ref):
        dis = lax.rsqrt(d_ref[...])
        h = jnp.maximum(
            dis * (a_ref[0] + a_ref[1] + y_ref[...]) + b_ref[...], 0.0
        )
        o_ref[...] = dis * jnp.dot(
            h, w_ref[...], preferred_element_type=jnp.float32
        )

    return pl.pallas_call(
        body,
        grid=(N // _BR,),
        in_specs=[
            pl.BlockSpec((NC, _BR, D), lambda i: (0, i, 0)),
            pl.BlockSpec((_BR, D), lambda i: (i, 0)),
            pl.BlockSpec((_BR, 1), lambda i: (i, 0)),
            pl.BlockSpec((1, D), lambda i: (0, 0)),
            pl.BlockSpec((D, Dout), lambda i: (0, 0)),
        ],
        out_specs=pl.BlockSpec((_BR, Dout), lambda i: (i, 0)),
        out_shape=jax.ShapeDtypeStruct((N, Dout), jnp.float32),
    )(aggp, y1, deg2, b1, W2)


def _fin(aggp, y2, deg2, b2):
    N, D = y2.shape

    def body(a_ref, y_ref, d_ref, b_ref, o_ref):
        dis = lax.rsqrt(d_ref[...])
        o_ref[...] = dis * (a_ref[0] + a_ref[1] + y_ref[...]) + b_ref[...]

    return pl.pallas_call(
        body,
        grid=(N // _BR,),
        in_specs=[
            pl.BlockSpec((NC, _BR, D), lambda i: (0, i, 0)),
            pl.BlockSpec((_BR, D), lambda i: (i, 0)),
            pl.BlockSpec((_BR, 1), lambda i: (i, 0)),
            pl.BlockSpec((1, D), lambda i: (0, 0)),
        ],
        out_specs=pl.BlockSpec((_BR, D), lambda i: (i, 0)),
        out_shape=jax.ShapeDtypeStruct((N, D), jnp.float32),
    )(aggp, y2, deg2, b2)


def kernel(x, edge_index, W1, b1, W2, b2):
    N, _ = x.shape
    E = edge_index.shape[1]
    D = W1.shape[1]
    NP = _pad_rows(N)

    src = edge_index[0].astype(jnp.int32)
    dst = edge_index[1].astype(jnp.int32)

    EP = NW * C * (-(-E // (NW * C)))
    if EP != E:
        pad = EP - E
        src = jnp.concatenate([src, jnp.zeros((pad,), jnp.int32)])
        dst = jnp.concatenate([dst, jnp.full((pad,), NP - 1, jnp.int32)])
    degp = _make_deg_kernel(EP, N)(dst)
    deg = degp.reshape(NC, NP, 128)[:, :N, 0].sum(0) + 1.0
    deg2 = deg.reshape(N, 1)

    y1 = _mm_scale(x, W1, deg2)
    agg1 = _make_agg_kernel(EP, N, D)(y1, src, dst).reshape(NC, NP, D)
    y2 = _mid(agg1, y1, deg2, b1.reshape(1, D), W2)
    agg2 = _make_agg_kernel(EP, N, D)(y2, src, dst).reshape(NC, NP, D)
    return _fin(agg2, y2, deg2, b2.reshape(1, D))
